# Initial kernel scaffold; baseline (speedup 1.0000x reference)
import jax
import jax.numpy as jnp
from jax.experimental import pallas as pl


def kernel(text, emb_table, fc_w, fc_b):
    raise NotImplementedError("write your pallas kernel here")


# speedup vs baseline: 52.4624x; 52.4624x over previous
import jax
import jax.numpy as jnp
from jax import lax
from jax.experimental import pallas as pl
from jax.experimental.pallas import tpu as pltpu
from jax.experimental.pallas import tpu_sc as plsc

VOCAB = 1000
EMBED = 64
NUM_CLASS = 10
BATCH = 16384
HIST = 200

CPAD = 16
NC, NS, LANES = 2, 16, 16
NW = NC * NS
SPW = BATCH // NW
CHUNK = 64
NCHUNK = SPW // CHUNK
GRP = CHUNK // LANES


def _p_body(emb_ref, wt_ref, b_ref, out_ref):
    p = jnp.dot(emb_ref[...], wt_ref[...], preferred_element_type=jnp.float32)
    out_ref[...] = (p + b_ref[...]) * (1.0 / HIST)


def _make_p(emb_table, wt_pad, b_pad):
    return pl.pallas_call(
        _p_body,
        out_shape=jax.ShapeDtypeStruct((VOCAB, CPAD), jnp.float32),
    )(emb_table, wt_pad, b_pad)


def _sc_body(p_hbm, text_hbm, out_hbm, p_v, text_v, out_v):
    wid = lax.axis_index("s") * NC + lax.axis_index("c")
    base = wid * SPW
    lanes = lax.iota(jnp.int32, 16)
    cvecs = [jnp.full((16,), c, jnp.int32) for c in range(NUM_CLASS)]

    pltpu.sync_copy(p_hbm, p_v)

    for ck in range(NCHUNK):
        pltpu.sync_copy(
            text_hbm.at[pl.ds((base + ck * CHUNK) * HIST, CHUNK * HIST)],
            text_v,
        )
        for g in range(GRP):
            tbase = (jnp.full((16,), g * LANES, jnp.int32) + lanes) * HIST

            def jbody(j, accs, tbase=tbase):
                idx = plsc.load_gather(text_v, [tbase + j])
                pidx = idx * CPAD
                return tuple(
                    accs[c] + plsc.load_gather(p_v, [pidx + cvecs[c]])
                    for c in range(NUM_CLASS)
                )

            accs = lax.fori_loop(
                0, HIST, jbody,
                tuple(jnp.zeros((16,), jnp.float32) for _ in range(NUM_CLASS)),
            )

            m = accs[0]
            for c in range(1, NUM_CLASS):
                m = jnp.maximum(m, accs[c])
            es = [jnp.exp(a - m) for a in accs]
            tot = es[0]
            for c in range(1, NUM_CLASS):
                tot = tot + es[c]
            obase = (jnp.full((16,), ck * CHUNK + g * LANES, jnp.int32) + lanes) * CPAD
            for c in range(NUM_CLASS):
                plsc.store_scatter(out_v, [obase + cvecs[c]], es[c] / tot)

    pltpu.sync_copy(out_v, out_hbm.at[pl.ds(base * CPAD, SPW * CPAD)])


_sc_call = pl.kernel(
    _sc_body,
    out_type=jax.ShapeDtypeStruct((BATCH * CPAD,), jnp.float32),
    mesh=plsc.VectorSubcoreMesh(core_axis_name="c", subcore_axis_name="s"),
    scratch_types=[
        pltpu.VMEM((VOCAB * CPAD,), jnp.float32),
        pltpu.VMEM((CHUNK * HIST,), jnp.int32),
        pltpu.VMEM((SPW * CPAD,), jnp.float32),
    ],
    compiler_params=pltpu.CompilerParams(
        use_tc_tiling_on_sc=False, needs_layout_passes=False
    ),
)


def kernel(text, emb_table, fc_w, fc_b):
    text = text.astype(jnp.int32)
    wt_pad = jnp.zeros((EMBED, CPAD), jnp.float32).at[:, :NUM_CLASS].set(fc_w.T)
    b_pad = jnp.zeros((1, CPAD), jnp.float32).at[0, :NUM_CLASS].set(fc_b)
    p = _make_p(emb_table, wt_pad, b_pad)
    out = _sc_call(p.reshape(-1), text.reshape(-1))
    return out.reshape(BATCH, CPAD)[:, :NUM_CLASS]


# speedup vs baseline: 94.0567x; 1.7928x over previous
import jax
import jax.numpy as jnp
from jax import lax
from jax.experimental import pallas as pl
from jax.experimental.pallas import tpu as pltpu
from jax.experimental.pallas import tpu_sc as plsc

VOCAB = 1000
EMBED = 64
NUM_CLASS = 10
BATCH = 16384
HIST = 200

CPAD = 16
NC, NS, LANES = 2, 16, 16
NW = NC * NS
SPW = BATCH // NW
CHUNK = 64
NCHUNK = SPW // CHUNK
GRP = CHUNK // LANES


def _p_body(emb_ref, w_ref, b_ref, out_ref):
    p = jax.lax.dot_general(
        w_ref[...], emb_ref[...],
        (((1,), (1,)), ((), ())),
        preferred_element_type=jnp.float32,
    )
    out_ref[...] = (p + b_ref[...]) * (1.0 / HIST)


def _make_p(emb_table, w_pad, b_pad):
    return pl.pallas_call(
        _p_body,
        out_shape=jax.ShapeDtypeStruct((CPAD, VOCAB), jnp.float32),
    )(emb_table, w_pad, b_pad)


def _sc_body(p_hbm, text_hbm, out_hbm, p_v, text_v, out_v):
    wid = lax.axis_index("s") * NC + lax.axis_index("c")
    base = wid * SPW
    lanes = lax.iota(jnp.int32, 16)
    cvecs = [jnp.full((16,), c, jnp.int32) for c in range(NUM_CLASS)]
    p_views = [p_v.at[pl.ds(c * VOCAB, VOCAB)] for c in range(NUM_CLASS)]
    UNROLL = 4

    pltpu.sync_copy(p_hbm, p_v)

    for ck in range(NCHUNK):
        pltpu.sync_copy(
            text_hbm.at[pl.ds((base + ck * CHUNK) * HIST, CHUNK * HIST)],
            text_v,
        )
        for g in range(GRP):
            tbase = (jnp.full((16,), g * LANES, jnp.int32) + lanes) * HIST

            def jbody(jb, accs, tbase=tbase):
                j0 = jb * UNROLL
                idxs = [
                    plsc.load_gather(text_v, [tbase + (j0 + u)])
                    for u in range(UNROLL)
                ]
                accs = list(accs)
                for u in range(UNROLL):
                    for c in range(NUM_CLASS):
                        accs[c] = accs[c] + plsc.load_gather(
                            p_views[c], [idxs[u]]
                        )
                return tuple(accs)

            accs = lax.fori_loop(
                0, HIST // UNROLL, jbody,
                tuple(jnp.zeros((16,), jnp.float32) for _ in range(NUM_CLASS)),
            )

            m = accs[0]
            for c in range(1, NUM_CLASS):
                m = jnp.maximum(m, accs[c])
            es = [jnp.exp(a - m) for a in accs]
            tot = es[0]
            for c in range(1, NUM_CLASS):
                tot = tot + es[c]
            obase = (jnp.full((16,), ck * CHUNK + g * LANES, jnp.int32) + lanes) * CPAD
            for c in range(NUM_CLASS):
                plsc.store_scatter(out_v, [obase + cvecs[c]], es[c] / tot)

    pltpu.sync_copy(out_v, out_hbm.at[pl.ds(base * CPAD, SPW * CPAD)])


_sc_call = pl.kernel(
    _sc_body,
    out_type=jax.ShapeDtypeStruct((BATCH * CPAD,), jnp.float32),
    mesh=plsc.VectorSubcoreMesh(core_axis_name="c", subcore_axis_name="s"),
    scratch_types=[
        pltpu.VMEM((VOCAB * CPAD,), jnp.float32),
        pltpu.VMEM((CHUNK * HIST,), jnp.int32),
        pltpu.VMEM((SPW * CPAD,), jnp.float32),
    ],
    compiler_params=pltpu.CompilerParams(
        use_tc_tiling_on_sc=False, needs_layout_passes=False
    ),
)


def kernel(text, emb_table, fc_w, fc_b):
    text = text.astype(jnp.int32)
    w_pad = jnp.zeros((CPAD, EMBED), jnp.float32).at[:NUM_CLASS].set(fc_w)
    b_pad = jnp.zeros((CPAD, 1), jnp.float32).at[:NUM_CLASS, 0].set(fc_b)
    p = _make_p(emb_table, w_pad, b_pad)
    out = _sc_call(p.reshape(-1), text.reshape(-1))
    return out.reshape(BATCH, CPAD)[:, :NUM_CLASS]


# speedup vs baseline: 96.2022x; 1.0228x over previous
import jax
import jax.numpy as jnp
from jax import lax
from jax.experimental import pallas as pl
from jax.experimental.pallas import tpu as pltpu
from jax.experimental.pallas import tpu_sc as plsc

VOCAB = 1000
EMBED = 64
NUM_CLASS = 10
BATCH = 16384
HIST = 200

CPAD = 16
NC, NS, LANES = 2, 16, 16
NW = NC * NS
SPW = BATCH // NW
CHUNK = 64
NCHUNK = SPW // CHUNK
GRP = CHUNK // LANES


def _p_body(emb_ref, w_ref, b_ref, out_ref):
    p = jax.lax.dot_general(
        w_ref[...], emb_ref[...],
        (((1,), (1,)), ((), ())),
        preferred_element_type=jnp.float32,
    )
    out_ref[...] = (p + b_ref[...]) * (1.0 / HIST)


def _make_p(emb_table, w_pad, b_pad):
    return pl.pallas_call(
        _p_body,
        out_shape=jax.ShapeDtypeStruct((CPAD, VOCAB), jnp.float32),
    )(emb_table, w_pad, b_pad)


def _sc_body(p_hbm, text_hbm, out_hbm, p_v, text_v, out_v):
    wid = lax.axis_index("s") * NC + lax.axis_index("c")
    base = wid * SPW
    lanes = lax.iota(jnp.int32, 16)
    cvecs = [jnp.full((16,), c, jnp.int32) for c in range(NUM_CLASS)]
    p_views = [p_v.at[pl.ds(c * VOCAB, VOCAB)] for c in range(NUM_CLASS)]
    UNROLL = 8

    pltpu.sync_copy(p_hbm, p_v)

    for ck in range(NCHUNK):
        pltpu.sync_copy(
            text_hbm.at[pl.ds((base + ck * CHUNK) * HIST, CHUNK * HIST)],
            text_v,
        )
        for g in range(GRP):
            tbase = (jnp.full((16,), g * LANES, jnp.int32) + lanes) * HIST

            def jbody(jb, accs, tbase=tbase):
                j0 = jb * UNROLL
                idxs = [
                    plsc.load_gather(text_v, [tbase + (j0 + u)])
                    for u in range(UNROLL)
                ]
                accs = list(accs)
                for u in range(UNROLL):
                    for c in range(NUM_CLASS):
                        accs[c] = accs[c] + plsc.load_gather(
                            p_views[c], [idxs[u]]
                        )
                return tuple(accs)

            accs = lax.fori_loop(
                0, HIST // UNROLL, jbody,
                tuple(jnp.zeros((16,), jnp.float32) for _ in range(NUM_CLASS)),
            )

            m = accs[0]
            for c in range(1, NUM_CLASS):
                m = jnp.maximum(m, accs[c])
            es = [jnp.exp(a - m) for a in accs]
            tot = es[0]
            for c in range(1, NUM_CLASS):
                tot = tot + es[c]
            obase = (jnp.full((16,), ck * CHUNK + g * LANES, jnp.int32) + lanes) * CPAD
            for c in range(NUM_CLASS):
                plsc.store_scatter(out_v, [obase + cvecs[c]], es[c] / tot)

    pltpu.sync_copy(out_v, out_hbm.at[pl.ds(base * CPAD, SPW * CPAD)])


_sc_call = pl.kernel(
    _sc_body,
    out_type=jax.ShapeDtypeStruct((BATCH * CPAD,), jnp.float32),
    mesh=plsc.VectorSubcoreMesh(core_axis_name="c", subcore_axis_name="s"),
    scratch_types=[
        pltpu.VMEM((VOCAB * CPAD,), jnp.float32),
        pltpu.VMEM((CHUNK * HIST,), jnp.int32),
        pltpu.VMEM((SPW * CPAD,), jnp.float32),
    ],
    compiler_params=pltpu.CompilerParams(
        use_tc_tiling_on_sc=False, needs_layout_passes=False
    ),
)


def kernel(text, emb_table, fc_w, fc_b):
    text = text.astype(jnp.int32)
    w_pad = jnp.zeros((CPAD, EMBED), jnp.float32).at[:NUM_CLASS].set(fc_w)
    b_pad = jnp.zeros((CPAD, 1), jnp.float32).at[:NUM_CLASS, 0].set(fc_b)
    p = _make_p(emb_table, w_pad, b_pad)
    out = _sc_call(p.reshape(-1), text.reshape(-1))
    return out.reshape(BATCH, CPAD)[:, :NUM_CLASS]
